# Initial kernel scaffold; baseline (speedup 1.0000x reference)
import jax
import jax.numpy as jnp
from jax.experimental import pallas as pl


def kernel(inputs, cluster_centers):
    raise NotImplementedError("write your pallas kernel here")


# speedup vs baseline: 1.7613x; 1.7613x over previous
import functools

import jax
import jax.numpy as jnp
from jax import lax
from jax.experimental import pallas as pl
from jax.experimental.pallas import tpu as pltpu
from jax.experimental.pallas import tpu_sc as plsc

_B = 4096
_K = 8192
_L = 16
_NC = 2
_NS = 16
_NW = _NC * _NS
_PTS = _B // _NW
_STEPS = 8
_INF = float("inf")


def _sc_body(x0_hbm, x1_hbm, c0s_hbm, c1s_hbm, ord_hbm, out_hbm,
             c0_v, c1_v, ord_v, x0_v, x1_v, out_v):
    wid = lax.axis_index("c") * _NS + lax.axis_index("s")
    base = wid * _PTS
    pltpu.sync_copy(c0s_hbm, c0_v)
    pltpu.sync_copy(c1s_hbm, c1_v)
    pltpu.sync_copy(ord_hbm, ord_v)
    pltpu.sync_copy(x0_hbm.at[pl.ds(base, _PTS)], x0_v)
    pltpu.sync_copy(x1_hbm.at[pl.ds(base, _PTS)], x1_v)

    zero = jnp.zeros((_L,), jnp.int32)

    def frontier(l, r, x0v):
        lc = jnp.maximum(l, zero)
        rc = jnp.minimum(r, jnp.int32(_K - 1))
        cl = plsc.load_gather(c0_v, [lc])
        cr = plsc.load_gather(c0_v, [rc])
        tl = x0v - cl
        tr = x0v - cr
        dl = jnp.where(l >= 0, tl * tl, _INF)
        dr = jnp.where(r <= _K - 1, tr * tr, _INF)
        return lc, rc, dl, dr

    for g in range(_PTS // _L):
        x0v = x0_v[pl.ds(g * _L, _L)]
        x1v = x1_v[pl.ds(g * _L, _L)]

        lo = zero
        hi = jnp.full((_L,), _K, jnp.int32)

        def bs_body(_, carry):
            lo, hi = carry
            mid = (lo + hi) >> 1
            cm = plsc.load_gather(c0_v, [mid])
            m = cm < x0v
            return jnp.where(m, mid + 1, lo), jnp.where(m, hi, mid)

        lo, hi = lax.fori_loop(0, 13, bs_body, (lo, hi))
        pos = lo

        def eval_side(idx_c, dxx, inb, best, bidx):
            c1j = plsc.load_gather(c1_v, [idx_c])
            ojf = plsc.load_gather(ord_v, [idx_c])
            oj = ojf
            t1 = x1v - c1j
            d = dxx + t1 * t1
            better = (d < best) | ((d == best) & (oj < bidx))
            better = better & inb
            return jnp.where(better, d, best), jnp.where(better, oj, bidx)

        def wbody(carry):
            l, r, best, bidx, _ = carry
            for _step in range(_STEPS):
                lc, rc, dl, dr = frontier(l, r, x0v)
                best, bidx = eval_side(lc, dl, l >= 0, best, bidx)
                best, bidx = eval_side(rc, dr, r <= _K - 1, best, bidx)
                l = l - 1
                r = r + 1
            _, _, dl, dr = frontier(l, r, x0v)
            livecnt = plsc.all_reduce_population_count(
                jnp.minimum(dl, dr) <= best)
            return l, r, best, bidx, livecnt[0]

        def wcond(carry):
            return carry[4] > 0

        init = (pos - 1, pos, jnp.full((_L,), _INF, jnp.float32), zero,
                jnp.int32(1))
        _, _, _, bidx, _ = lax.while_loop(wcond, wbody, init)
        out_v[pl.ds(g * _L, _L)] = bidx

    pltpu.sync_copy(out_v, out_hbm.at[pl.ds(base, _PTS)])


@functools.partial(jax.jit, static_argnums=())
def kernel(inputs, cluster_centers):
    x0 = inputs[:, 0]
    x1 = inputs[:, 1]
    c0 = cluster_centers[:, 0]
    c1 = cluster_centers[:, 1]
    order = jnp.argsort(c0)
    c0s = c0[order]
    c1s = c1[order]
    run = pl.kernel(
        _sc_body,
        out_type=jax.ShapeDtypeStruct((_B,), jnp.int32),
        mesh=plsc.VectorSubcoreMesh(core_axis_name="c", subcore_axis_name="s"),
        compiler_params=pltpu.CompilerParams(needs_layout_passes=False),
        scratch_types=[
            pltpu.VMEM((_K,), jnp.float32),
            pltpu.VMEM((_K,), jnp.float32),
            pltpu.VMEM((_K,), jnp.int32),
            pltpu.VMEM((_PTS,), jnp.float32),
            pltpu.VMEM((_PTS,), jnp.float32),
            pltpu.VMEM((_PTS,), jnp.int32),
        ],
    )
    return run(x0, x1, c0s, c1s, order)


# speedup vs baseline: 2.4511x; 1.3917x over previous
import functools

import jax
import jax.numpy as jnp
from jax import lax
from jax.experimental import pallas as pl
from jax.experimental.pallas import tpu as pltpu
from jax.experimental.pallas import tpu_sc as plsc

_B = 4096
_K = 8192
_L = 16
_NC = 2
_NS = 16
_NW = _NC * _NS
_PTS = _B // _NW
_STEPS = 16
_PAD = 16
_SZ = _K + 2 * _PAD
_RMAX = _SZ - 1
_BIG = 3.0e18
_INF = float("inf")


def _sc_body(x0_hbm, x1_hbm, c0s_hbm, c1s_hbm, ord_hbm, out_hbm,
             c0_v, c1_v, ord_v, x0_v, x1_v, out_v):
    wid = lax.axis_index("c") * _NS + lax.axis_index("s")
    base = wid * _PTS
    pltpu.sync_copy(c0s_hbm, c0_v.at[pl.ds(_PAD, _K)])
    pltpu.sync_copy(c1s_hbm, c1_v.at[pl.ds(_PAD, _K)])
    pltpu.sync_copy(ord_hbm, ord_v.at[pl.ds(_PAD, _K)])
    pltpu.sync_copy(x0_hbm.at[pl.ds(base, _PTS)], x0_v)
    pltpu.sync_copy(x1_hbm.at[pl.ds(base, _PTS)], x1_v)
    big = jnp.full((_PAD,), _BIG, jnp.float32)
    c0_v[pl.ds(0, _PAD)] = big
    c0_v[pl.ds(_PAD + _K, _PAD)] = big
    zeroi = jnp.zeros((_PAD,), jnp.int32)
    ord_v[pl.ds(0, _PAD)] = zeroi
    ord_v[pl.ds(_PAD + _K, _PAD)] = zeroi
    c1_v[pl.ds(0, _PAD)] = big
    c1_v[pl.ds(_PAD + _K, _PAD)] = big

    zero = jnp.zeros((_L,), jnp.int32)

    def frontier(l, r, x0v):
        cl = plsc.load_gather(c0_v, [l])
        cr = plsc.load_gather(c0_v, [r])
        tl = x0v - cl
        tr = x0v - cr
        return tl * tl, tr * tr

    for g in range(_PTS // _L):
        x0v = x0_v[pl.ds(g * _L, _L)]
        x1v = x1_v[pl.ds(g * _L, _L)]

        lo = jnp.full((_L,), _PAD, jnp.int32)
        hi = jnp.full((_L,), _PAD + _K, jnp.int32)

        def bs_body(_, carry):
            lo, hi = carry
            mid = (lo + hi) >> 1
            cm = plsc.load_gather(c0_v, [mid])
            m = cm < x0v
            return jnp.where(m, mid + 1, lo), jnp.where(m, hi, mid)

        lo, hi = lax.fori_loop(0, 13, bs_body, (lo, hi))
        pos = lo

        def eval_side(idx_c, dxx, best, bidx):
            c1j = plsc.load_gather(c1_v, [idx_c])
            oj = plsc.load_gather(ord_v, [idx_c])
            t1 = x1v - c1j
            d = dxx + t1 * t1
            better = (d < best) | ((d == best) & (oj < bidx))
            return jnp.where(better, d, best), jnp.where(better, oj, bidx)

        def wbody(carry):
            l, r, bl, il, br, ir, _ = carry
            for _step in range(_STEPS):
                dl, dr = frontier(l, r, x0v)
                bl, il = eval_side(l, dl, bl, il)
                br, ir = eval_side(r, dr, br, ir)
                l = jnp.maximum(l - 1, zero)
                r = jnp.minimum(r + 1, jnp.int32(_RMAX))
            dl, dr = frontier(l, r, x0v)
            live = jnp.minimum(dl, dr) <= jnp.minimum(bl, br)
            live = live & ~((l == 0) & (r == _RMAX))
            livecnt = plsc.all_reduce_population_count(live)
            return l, r, bl, il, br, ir, livecnt[0]

        def wcond(carry):
            return carry[6] > 0

        inf_v = jnp.full((_L,), _INF, jnp.float32)
        init = (jnp.maximum(pos - 1, zero), pos,
                inf_v, zero, inf_v, zero, jnp.int32(1))
        _, _, bl, il, br, ir, _ = lax.while_loop(wcond, wbody, init)
        rwins = (br < bl) | ((br == bl) & (ir < il))
        bidx = jnp.where(rwins, ir, il)
        out_v[pl.ds(g * _L, _L)] = bidx

    pltpu.sync_copy(out_v, out_hbm.at[pl.ds(base, _PTS)])


@functools.partial(jax.jit, static_argnums=())
def kernel(inputs, cluster_centers):
    x0 = inputs[:, 0]
    x1 = inputs[:, 1]
    c0 = cluster_centers[:, 0]
    c1 = cluster_centers[:, 1]
    iota = lax.iota(jnp.int32, _K)
    c0s, c1s, order = lax.sort((c0, c1, iota), num_keys=1, is_stable=False)
    run = pl.kernel(
        _sc_body,
        out_type=jax.ShapeDtypeStruct((_B,), jnp.int32),
        mesh=plsc.VectorSubcoreMesh(core_axis_name="c", subcore_axis_name="s"),
        compiler_params=pltpu.CompilerParams(needs_layout_passes=False),
        scratch_types=[
            pltpu.VMEM((_SZ,), jnp.float32),
            pltpu.VMEM((_SZ,), jnp.float32),
            pltpu.VMEM((_SZ,), jnp.int32),
            pltpu.VMEM((_PTS,), jnp.float32),
            pltpu.VMEM((_PTS,), jnp.float32),
            pltpu.VMEM((_PTS,), jnp.int32),
        ],
    )
    return run(x0, x1, c0s, c1s, order)
